# baseline (speedup 1.0000x reference)
import functools

import jax
import jax.numpy as jnp
from jax import lax
from jax.experimental import pallas as pl
from jax.experimental.pallas import tpu as pltpu
from jax.experimental.pallas import tpu_sc as plsc

N = 4096
M = 16384
K = 8
NC, NS, L = 2, 16, 16
NW = NC * NS
QPW = N // NW
OUTC = 16
BIG = 3.4e38


def _bitonic_sort_body(xm_ref, out_ref):
    v = xm_ref[...]
    r = lax.broadcasted_iota(jnp.int32, (128, 128), 0)
    c = lax.broadcasted_iota(jnp.int32, (128, 128), 1)
    idx = c * 128 + r
    for lm in range(1, 15):
        mm = 1 << lm
        desc = (idx & mm) != 0
        for ls in range(lm - 1, -1, -1):
            s = 1 << ls
            bit = (idx & s) != 0
            if s < 128:
                up = jnp.roll(v, -s, axis=0)
                dn = jnp.roll(v, s, axis=0)
            else:
                up = jnp.roll(v, -(s // 128), axis=1)
                dn = jnp.roll(v, s // 128, axis=1)
            partner = jnp.where(bit, dn, up)
            lo = jnp.minimum(v, partner)
            hi = jnp.maximum(v, partner)
            v = jnp.where(bit != desc, hi, lo)
    out_ref[...] = v.T


_tc_sort = pl.pallas_call(
    _bitonic_sort_body,
    out_shape=jax.ShapeDtypeStruct((128, 128), jnp.float32),
)


def _sc_query_body(xs_hbm, x_hbm, out_hbm, xs_v, q_v, out_v, sem):
    wid = lax.axis_index("s") * NC + lax.axis_index("c")
    base = wid * QPW
    nst = 4
    cps = [
        pltpu.async_copy(
            xs_hbm.at[pl.ds(i * (M // nst), M // nst)],
            xs_v.at[pl.ds(i * (M // nst), M // nst)], sem)
        for i in range(nst)
    ]
    pltpu.sync_copy(x_hbm.at[pl.ds(base, QPW)], q_v)
    for cp in cps:
        cp.wait()
    lanes = lax.iota(jnp.int32, L)
    ng = QPW // L
    qs = [q_v[pl.ds(g * L, L)] for g in range(ng)]
    def _bs_step(_, carry):
        los, his = carry
        mids = [(los[g] + his[g]) >> 1 for g in range(ng)]
        vals = [plsc.load_gather(xs_v, [mids[g]]) for g in range(ng)]
        nlo, nhi = [], []
        for g in range(ng):
            pred = vals[g] < qs[g]
            nlo.append(jnp.where(pred, mids[g] + 1, los[g]))
            nhi.append(jnp.where(pred, his[g], mids[g]))
        return tuple(nlo), tuple(nhi)

    los, his = lax.fori_loop(
        0, 14, _bs_step,
        (tuple(jnp.zeros((L,), jnp.int32) for _ in range(ng)),
         tuple(jnp.full((L,), M, jnp.int32) for _ in range(ng))))
    los = list(los)
    lefts = [los[g] - 1 for g in range(ng)]
    rights = los
    dls, drs = [], []
    for g in range(ng):
        lval = plsc.load_gather(xs_v, [jnp.maximum(lefts[g], 0)])
        rval = plsc.load_gather(xs_v, [jnp.minimum(rights[g], M - 1)])
        dls.append(jnp.where(lefts[g] >= 0, jnp.abs(qs[g] - lval), BIG))
        drs.append(jnp.where(rights[g] < M, jnp.abs(qs[g] - rval), BIG))
        plsc.store_scatter(
            out_v, [lanes + g * L, jnp.zeros((L,), jnp.int32)], qs[g])
    for t in range(1, K + 1):
        col = jnp.zeros((L,), jnp.int32) + t
        takes = [dls[g] <= drs[g] for g in range(ng)]
        for g in range(ng):
            plsc.store_scatter(
                out_v, [lanes + g * L, col],
                jnp.where(takes[g], dls[g], drs[g]))
        if t == K:
            break
        for g in range(ng):
            lefts[g] = jnp.where(takes[g], lefts[g] - 1, lefts[g])
            rights[g] = jnp.where(takes[g], rights[g], rights[g] + 1)
        moved = [jnp.where(takes[g], lefts[g], rights[g]) for g in range(ng)]
        vals = [
            plsc.load_gather(xs_v, [jnp.clip(moved[g], 0, M - 1)])
            for g in range(ng)
        ]
        for g in range(ng):
            valid = jnp.where(takes[g], lefts[g] >= 0, rights[g] < M)
            nd = jnp.where(valid, jnp.abs(qs[g] - vals[g]), BIG)
            dls[g] = jnp.where(takes[g], nd, dls[g])
            drs[g] = jnp.where(takes[g], drs[g], nd)
    pltpu.sync_copy(out_v, out_hbm.at[pl.ds(base, QPW)])


@functools.lru_cache(maxsize=1)
def _make_sc_query():
    return pl.kernel(
        _sc_query_body,
        out_type=jax.ShapeDtypeStruct((N, K + 1), jnp.float32),
        mesh=plsc.VectorSubcoreMesh(
            core_axis_name="c", subcore_axis_name="s",
            num_cores=NC, num_subcores=NS),
        scratch_types=[
            pltpu.VMEM((M,), jnp.float32),
            pltpu.VMEM((QPW,), jnp.float32),
            pltpu.VMEM((QPW, K + 1), jnp.float32),
            pltpu.SemaphoreType.DMA,
        ],
        compiler_params=pltpu.CompilerParams(needs_layout_passes=False),
    )


@jax.jit
def kernel(x, x_measured):
    xs = _tc_sort(x_measured.reshape(128, 128)).reshape(M)
    return _make_sc_query()(xs, x)


# speedup vs baseline: 1.0794x; 1.0794x over previous
import functools

import jax
import jax.numpy as jnp
from jax import lax
from jax.experimental import pallas as pl
from jax.experimental.pallas import tpu as pltpu
from jax.experimental.pallas import tpu_sc as plsc

N = 4096
M = 16384
K = 8
NC, NS, L = 2, 16, 16
NW = NC * NS
QPW = N // NW
OUTC = 16
BIG = 3.4e38


def _bitonic_sort_body(xm_ref, out_ref):
    v = xm_ref[...]
    r = lax.broadcasted_iota(jnp.int32, (128, 128), 0)
    c = lax.broadcasted_iota(jnp.int32, (128, 128), 1)
    idx = c * 128 + r
    for lm in range(1, 15):
        mm = 1 << lm
        desc = (idx & mm) != 0
        for ls in range(lm - 1, -1, -1):
            s = 1 << ls
            bit = (idx & s) != 0
            if s < 128:
                up = jnp.roll(v, -s, axis=0)
                dn = jnp.roll(v, s, axis=0)
            else:
                up = jnp.roll(v, -(s // 128), axis=1)
                dn = jnp.roll(v, s // 128, axis=1)
            partner = jnp.where(bit, dn, up)
            lo = jnp.minimum(v, partner)
            hi = jnp.maximum(v, partner)
            v = jnp.where(bit != desc, hi, lo)
    out_ref[...] = v.T


_tc_sort = pl.pallas_call(
    _bitonic_sort_body,
    out_shape=jax.ShapeDtypeStruct((128, 128), jnp.float32),
)


def _sc_query_body(xs_hbm, x_hbm, out_hbm, xs_v, q_v, out_v, xs_sh, sem):
    sid = lax.axis_index("s")
    wid = sid * NC + lax.axis_index("c")
    base = wid * QPW
    @pl.when(sid == 0)
    def _stage_shared():
        pltpu.sync_copy(xs_hbm, xs_sh)
    plsc.subcore_barrier()
    nst = 4
    cps = [
        pltpu.async_copy(
            xs_sh.at[pl.ds(i * (M // nst), M // nst)],
            xs_v.at[pl.ds(i * (M // nst), M // nst)], sem)
        for i in range(nst)
    ]
    pltpu.sync_copy(x_hbm.at[pl.ds(base, QPW)], q_v)
    for cp in cps:
        cp.wait()
    lanes = lax.iota(jnp.int32, L)
    ng = QPW // L
    qs = [q_v[pl.ds(g * L, L)] for g in range(ng)]
    def _bs_step(_, carry):
        los, his = carry
        mids = [(los[g] + his[g]) >> 1 for g in range(ng)]
        vals = [plsc.load_gather(xs_v, [mids[g]]) for g in range(ng)]
        nlo, nhi = [], []
        for g in range(ng):
            pred = vals[g] < qs[g]
            nlo.append(jnp.where(pred, mids[g] + 1, los[g]))
            nhi.append(jnp.where(pred, his[g], mids[g]))
        return tuple(nlo), tuple(nhi)

    los, his = lax.fori_loop(
        0, 14, _bs_step,
        (tuple(jnp.zeros((L,), jnp.int32) for _ in range(ng)),
         tuple(jnp.full((L,), M, jnp.int32) for _ in range(ng))))
    los = list(los)
    lefts = [los[g] - 1 for g in range(ng)]
    rights = los
    dls, drs = [], []
    for g in range(ng):
        lval = plsc.load_gather(xs_v, [jnp.maximum(lefts[g], 0)])
        rval = plsc.load_gather(xs_v, [jnp.minimum(rights[g], M - 1)])
        dls.append(jnp.where(lefts[g] >= 0, jnp.abs(qs[g] - lval), BIG))
        drs.append(jnp.where(rights[g] < M, jnp.abs(qs[g] - rval), BIG))
        plsc.store_scatter(
            out_v, [lanes + g * L, jnp.zeros((L,), jnp.int32)], qs[g])
    def _merge_step(t, carry):
        lefts, rights, dls, drs = (list(c) for c in carry)
        col = jnp.zeros((L,), jnp.int32) + t
        takes = [dls[g] <= drs[g] for g in range(ng)]
        for g in range(ng):
            plsc.store_scatter(
                out_v, [lanes + g * L, col],
                jnp.where(takes[g], dls[g], drs[g]))
        for g in range(ng):
            lefts[g] = jnp.where(takes[g], lefts[g] - 1, lefts[g])
            rights[g] = jnp.where(takes[g], rights[g], rights[g] + 1)
        moved = [jnp.where(takes[g], lefts[g], rights[g]) for g in range(ng)]
        vals = [
            plsc.load_gather(xs_v, [jnp.clip(moved[g], 0, M - 1)])
            for g in range(ng)
        ]
        for g in range(ng):
            valid = jnp.where(takes[g], lefts[g] >= 0, rights[g] < M)
            nd = jnp.where(valid, jnp.abs(qs[g] - vals[g]), BIG)
            dls[g] = jnp.where(takes[g], nd, dls[g])
            drs[g] = jnp.where(takes[g], drs[g], nd)
        return tuple(lefts), tuple(rights), tuple(dls), tuple(drs)

    _, _, dls, drs = lax.fori_loop(
        1, K, _merge_step,
        (tuple(lefts), tuple(rights), tuple(dls), tuple(drs)))
    colk = jnp.zeros((L,), jnp.int32) + K
    for g in range(ng):
        plsc.store_scatter(
            out_v, [lanes + g * L, colk],
            jnp.where(dls[g] <= drs[g], dls[g], drs[g]))
    pltpu.sync_copy(out_v, out_hbm.at[pl.ds(base, QPW)])


@functools.lru_cache(maxsize=1)
def _make_sc_query():
    return pl.kernel(
        _sc_query_body,
        out_type=jax.ShapeDtypeStruct((N, K + 1), jnp.float32),
        mesh=plsc.VectorSubcoreMesh(
            core_axis_name="c", subcore_axis_name="s",
            num_cores=NC, num_subcores=NS),
        scratch_types=[
            pltpu.VMEM((M,), jnp.float32),
            pltpu.VMEM((QPW,), jnp.float32),
            pltpu.VMEM((QPW, K + 1), jnp.float32),
            pltpu.VMEM_SHARED((M,), jnp.float32),
            pltpu.SemaphoreType.DMA,
        ],
        compiler_params=pltpu.CompilerParams(needs_layout_passes=False),
    )


@jax.jit
def kernel(x, x_measured):
    xs = _tc_sort(x_measured.reshape(128, 128)).reshape(M)
    return _make_sc_query()(xs, x)


# speedup vs baseline: 1.0830x; 1.0034x over previous
import functools

import jax
import jax.numpy as jnp
from jax import lax
from jax.experimental import pallas as pl
from jax.experimental.pallas import tpu as pltpu
from jax.experimental.pallas import tpu_sc as plsc

N = 4096
M = 16384
K = 8
NC, NS, L = 2, 16, 16
NW = NC * NS
QPW = N // NW
OUTC = 16
BIG = 3.4e38


def _bitonic_sort_body(xm_ref, out_ref):
    v = xm_ref[...]
    r = lax.broadcasted_iota(jnp.int32, (128, 128), 0)
    c = lax.broadcasted_iota(jnp.int32, (128, 128), 1)
    idx = c * 128 + r
    for lm in range(1, 15):
        mm = 1 << lm
        desc = (idx & mm) != 0
        for ls in range(lm - 1, -1, -1):
            s = 1 << ls
            bit = (idx & s) != 0
            if s < 128:
                up = jnp.roll(v, -s, axis=0)
                dn = jnp.roll(v, s, axis=0)
            else:
                up = jnp.roll(v, -(s // 128), axis=1)
                dn = jnp.roll(v, s // 128, axis=1)
            partner = jnp.where(bit, dn, up)
            lo = jnp.minimum(v, partner)
            hi = jnp.maximum(v, partner)
            v = jnp.where(bit != desc, hi, lo)
    out_ref[...] = v.T


_tc_sort = pl.pallas_call(
    _bitonic_sort_body,
    out_shape=jax.ShapeDtypeStruct((128, 128), jnp.float32),
)


def _sc_query_body(xs_hbm, x_hbm, out_hbm, xs_v, q_v, out_v, xs_sh, sem):
    sid = lax.axis_index("s")
    wid = sid * NC + lax.axis_index("c")
    base = wid * QPW
    @pl.when(sid == 0)
    def _stage_shared():
        pltpu.sync_copy(xs_hbm, xs_sh)
    plsc.subcore_barrier()
    nst = 4
    cps = [
        pltpu.async_copy(
            xs_sh.at[pl.ds(i * (M // nst), M // nst)],
            xs_v.at[pl.ds(i * (M // nst), M // nst)], sem)
        for i in range(nst)
    ]
    pltpu.sync_copy(x_hbm.at[pl.ds(base, QPW)], q_v)
    for cp in cps:
        cp.wait()
    lanes = lax.iota(jnp.int32, L)
    ng = QPW // L
    qs = [q_v[pl.ds(g * L, L)] for g in range(ng)]
    def _bs_round(los, his):
        mids = [(los[g] + his[g]) >> 1 for g in range(ng)]
        vals = [plsc.load_gather(xs_v, [mids[g]]) for g in range(ng)]
        nlo, nhi = [], []
        for g in range(ng):
            pred = vals[g] < qs[g]
            nlo.append(jnp.where(pred, mids[g] + 1, los[g]))
            nhi.append(jnp.where(pred, his[g], mids[g]))
        return nlo, nhi

    def _bs_step(_, carry):
        los, his = (list(c) for c in carry)
        los, his = _bs_round(los, his)
        los, his = _bs_round(los, his)
        return tuple(los), tuple(his)

    los, his = lax.fori_loop(
        0, 7, _bs_step,
        (tuple(jnp.zeros((L,), jnp.int32) for _ in range(ng)),
         tuple(jnp.full((L,), M, jnp.int32) for _ in range(ng))))
    los = list(los)
    lefts = [los[g] - 1 for g in range(ng)]
    rights = los
    dls, drs = [], []
    for g in range(ng):
        lval = plsc.load_gather(xs_v, [jnp.maximum(lefts[g], 0)])
        rval = plsc.load_gather(xs_v, [jnp.minimum(rights[g], M - 1)])
        dls.append(jnp.where(lefts[g] >= 0, jnp.abs(qs[g] - lval), BIG))
        drs.append(jnp.where(rights[g] < M, jnp.abs(qs[g] - rval), BIG))
        plsc.store_scatter(
            out_v, [lanes + g * L, jnp.zeros((L,), jnp.int32)], qs[g])
    def _merge_step(t, carry):
        lefts, rights, dls, drs = (list(c) for c in carry)
        col = jnp.zeros((L,), jnp.int32) + t
        takes = [dls[g] <= drs[g] for g in range(ng)]
        for g in range(ng):
            plsc.store_scatter(
                out_v, [lanes + g * L, col],
                jnp.where(takes[g], dls[g], drs[g]))
        for g in range(ng):
            lefts[g] = jnp.where(takes[g], lefts[g] - 1, lefts[g])
            rights[g] = jnp.where(takes[g], rights[g], rights[g] + 1)
        moved = [jnp.where(takes[g], lefts[g], rights[g]) for g in range(ng)]
        vals = [
            plsc.load_gather(xs_v, [jnp.clip(moved[g], 0, M - 1)])
            for g in range(ng)
        ]
        for g in range(ng):
            valid = jnp.where(takes[g], lefts[g] >= 0, rights[g] < M)
            nd = jnp.where(valid, jnp.abs(qs[g] - vals[g]), BIG)
            dls[g] = jnp.where(takes[g], nd, dls[g])
            drs[g] = jnp.where(takes[g], drs[g], nd)
        return tuple(lefts), tuple(rights), tuple(dls), tuple(drs)

    _, _, dls, drs = lax.fori_loop(
        1, K, _merge_step,
        (tuple(lefts), tuple(rights), tuple(dls), tuple(drs)))
    colk = jnp.zeros((L,), jnp.int32) + K
    for g in range(ng):
        plsc.store_scatter(
            out_v, [lanes + g * L, colk],
            jnp.where(dls[g] <= drs[g], dls[g], drs[g]))
    pltpu.sync_copy(out_v, out_hbm.at[pl.ds(base, QPW)])


@functools.lru_cache(maxsize=1)
def _make_sc_query():
    return pl.kernel(
        _sc_query_body,
        out_type=jax.ShapeDtypeStruct((N, K + 1), jnp.float32),
        mesh=plsc.VectorSubcoreMesh(
            core_axis_name="c", subcore_axis_name="s",
            num_cores=NC, num_subcores=NS),
        scratch_types=[
            pltpu.VMEM((M,), jnp.float32),
            pltpu.VMEM((QPW,), jnp.float32),
            pltpu.VMEM((QPW, K + 1), jnp.float32),
            pltpu.VMEM_SHARED((M,), jnp.float32),
            pltpu.SemaphoreType.DMA,
        ],
        compiler_params=pltpu.CompilerParams(needs_layout_passes=False),
    )


@jax.jit
def kernel(x, x_measured):
    xs = _tc_sort(x_measured.reshape(128, 128)).reshape(M)
    return _make_sc_query()(xs, x)


# speedup vs baseline: 1.0908x; 1.0072x over previous
import functools

import jax
import jax.numpy as jnp
from jax import lax
from jax.experimental import pallas as pl
from jax.experimental.pallas import tpu as pltpu
from jax.experimental.pallas import tpu_sc as plsc

N = 4096
M = 16384
K = 8
NC, NS, L = 2, 16, 16
NW = NC * NS
QPW = N // NW
OUTC = 16
BIG = 3.4e38


def _bitonic_sort_body(xm_ref, out_ref):
    v = xm_ref[...]
    r = lax.broadcasted_iota(jnp.int32, (128, 128), 0)
    c = lax.broadcasted_iota(jnp.int32, (128, 128), 1)
    idx = c * 128 + r
    for lm in range(1, 15):
        mm = 1 << lm
        desc = (idx & mm) != 0
        for ls in range(lm - 1, -1, -1):
            s = 1 << ls
            bit = (idx & s) != 0
            if s < 128:
                up = jnp.roll(v, -s, axis=0)
                dn = jnp.roll(v, s, axis=0)
            else:
                up = jnp.roll(v, -(s // 128), axis=1)
                dn = jnp.roll(v, s // 128, axis=1)
            partner = jnp.where(bit, dn, up)
            lo = jnp.minimum(v, partner)
            hi = jnp.maximum(v, partner)
            v = jnp.where(bit != desc, hi, lo)
    out_ref[...] = v.T


_tc_sort = pl.pallas_call(
    _bitonic_sort_body,
    out_shape=jax.ShapeDtypeStruct((128, 128), jnp.float32),
)


def _sc_query_body(xs_hbm, x_hbm, out_hbm, xs_v, q_v, out_v, xs_sh, sem):
    sid = lax.axis_index("s")
    wid = sid * NC + lax.axis_index("c")
    base = wid * QPW
    @pl.when(sid == 0)
    def _stage_shared():
        pltpu.sync_copy(xs_hbm, xs_sh)
    plsc.subcore_barrier()
    nst = 4
    cps = [
        pltpu.async_copy(
            xs_sh.at[pl.ds(i * (M // nst), M // nst)],
            xs_v.at[pl.ds(i * (M // nst), M // nst)], sem)
        for i in range(nst)
    ]
    pltpu.sync_copy(x_hbm.at[pl.ds(base, QPW)], q_v)
    for cp in cps:
        cp.wait()
    lanes = lax.iota(jnp.int32, L)
    ng = QPW // L
    qs = [q_v[pl.ds(g * L, L)] for g in range(ng)]
    def _bs_round(cnts, s):
        idxs = [cnts[g] + (s - 1) for g in range(ng)]
        vals = [plsc.load_gather(xs_v, [idxs[g]]) for g in range(ng)]
        return [
            jnp.where(vals[g] < qs[g], cnts[g] + s, cnts[g])
            for g in range(ng)
        ]

    def _bs_step(i, cnts):
        cnts = list(cnts)
        cnts = _bs_round(cnts, M >> (2 * i + 1))
        cnts = _bs_round(cnts, M >> (2 * i + 2))
        return tuple(cnts)

    los = list(lax.fori_loop(
        0, 7, _bs_step,
        tuple(jnp.zeros((L,), jnp.int32) for _ in range(ng))))
    lefts = [los[g] - 1 for g in range(ng)]
    rights = los
    dls, drs = [], []
    for g in range(ng):
        lval = plsc.load_gather(xs_v, [jnp.maximum(lefts[g], 0)])
        rval = plsc.load_gather(xs_v, [jnp.minimum(rights[g], M - 1)])
        dls.append(jnp.where(lefts[g] >= 0, jnp.abs(qs[g] - lval), BIG))
        drs.append(jnp.where(rights[g] < M, jnp.abs(qs[g] - rval), BIG))
        plsc.store_scatter(
            out_v, [lanes + g * L, jnp.zeros((L,), jnp.int32)], qs[g])
    def _merge_step(t, carry):
        lefts, dls, drs = (list(c) for c in carry)
        col = jnp.zeros((L,), jnp.int32) + t
        takes = [dls[g] <= drs[g] for g in range(ng)]
        for g in range(ng):
            plsc.store_scatter(
                out_v, [lanes + g * L, col],
                jnp.where(takes[g], dls[g], drs[g]))
        for g in range(ng):
            lefts[g] = jnp.where(takes[g], lefts[g] - 1, lefts[g])
        rights = [lefts[g] + (t + 1) for g in range(ng)]
        moved = [jnp.where(takes[g], lefts[g], rights[g]) for g in range(ng)]
        vals = [
            plsc.load_gather(xs_v, [jnp.clip(moved[g], 0, M - 1)])
            for g in range(ng)
        ]
        for g in range(ng):
            valid = jnp.where(takes[g], lefts[g] >= 0, rights[g] < M)
            nd = jnp.where(valid, jnp.abs(qs[g] - vals[g]), BIG)
            dls[g] = jnp.where(takes[g], nd, dls[g])
            drs[g] = jnp.where(takes[g], drs[g], nd)
        return tuple(lefts), tuple(dls), tuple(drs)

    _, dls, drs = lax.fori_loop(
        1, K, _merge_step,
        (tuple(lefts), tuple(dls), tuple(drs)))
    colk = jnp.zeros((L,), jnp.int32) + K
    for g in range(ng):
        plsc.store_scatter(
            out_v, [lanes + g * L, colk],
            jnp.where(dls[g] <= drs[g], dls[g], drs[g]))
    pltpu.sync_copy(out_v, out_hbm.at[pl.ds(base, QPW)])


@functools.lru_cache(maxsize=1)
def _make_sc_query():
    return pl.kernel(
        _sc_query_body,
        out_type=jax.ShapeDtypeStruct((N, K + 1), jnp.float32),
        mesh=plsc.VectorSubcoreMesh(
            core_axis_name="c", subcore_axis_name="s",
            num_cores=NC, num_subcores=NS),
        scratch_types=[
            pltpu.VMEM((M,), jnp.float32),
            pltpu.VMEM((QPW,), jnp.float32),
            pltpu.VMEM((QPW, K + 1), jnp.float32),
            pltpu.VMEM_SHARED((M,), jnp.float32),
            pltpu.SemaphoreType.DMA,
        ],
        compiler_params=pltpu.CompilerParams(needs_layout_passes=False),
    )


@jax.jit
def kernel(x, x_measured):
    xs = _tc_sort(x_measured.reshape(128, 128)).reshape(M)
    return _make_sc_query()(xs, x)


# speedup vs baseline: 1.0911x; 1.0003x over previous
import functools

import jax
import jax.numpy as jnp
from jax import lax
from jax.experimental import pallas as pl
from jax.experimental.pallas import tpu as pltpu
from jax.experimental.pallas import tpu_sc as plsc

N = 4096
M = 16384
K = 8
NC, NS, L = 2, 16, 16
NW = NC * NS
QPW = N // NW
BIG = 3.4e38


def _bitonic_sort_body(xm_ref, out_ref):
    v = xm_ref[...]
    r = lax.broadcasted_iota(jnp.int32, (128, 128), 0)
    c = lax.broadcasted_iota(jnp.int32, (128, 128), 1)
    idx = c * 128 + r
    for lm in range(1, 15):
        mm = 1 << lm
        desc = (idx & mm) != 0
        for ls in range(lm - 1, -1, -1):
            s = 1 << ls
            bit = (idx & s) != 0
            if s < 128:
                up = jnp.roll(v, -s, axis=0)
                dn = jnp.roll(v, s, axis=0)
            else:
                up = jnp.roll(v, -(s // 128), axis=1)
                dn = jnp.roll(v, s // 128, axis=1)
            partner = jnp.where(bit, dn, up)
            lo = jnp.minimum(v, partner)
            hi = jnp.maximum(v, partner)
            v = jnp.where(bit != desc, hi, lo)
    out_ref[...] = v.T


_tc_sort = pl.pallas_call(
    _bitonic_sort_body,
    out_shape=jax.ShapeDtypeStruct((128, 128), jnp.float32),
)


def _sc_query_body(xs_hbm, x_hbm, out_hbm, xs_v, q_v, out_v, xs_sh, sem):
    sid = lax.axis_index("s")
    wid = sid * NC + lax.axis_index("c")
    base = wid * QPW
    @pl.when(sid == 0)
    def _stage_shared():
        pltpu.sync_copy(xs_hbm, xs_sh)
    plsc.subcore_barrier()
    nst = 4
    cps = [
        pltpu.async_copy(
            xs_sh.at[pl.ds(i * (M // nst), M // nst)],
            xs_v.at[pl.ds(i * (M // nst), M // nst)], sem)
        for i in range(nst)
    ]
    pltpu.sync_copy(x_hbm.at[pl.ds(base, QPW)], q_v)
    for cp in cps:
        cp.wait()
    lanes = lax.iota(jnp.int32, L)
    ng = QPW // L
    qs = [q_v[pl.ds(g * L, L)] for g in range(ng)]
    def _bs_round(cnts, s):
        idxs = [cnts[g] + (s - 1) for g in range(ng)]
        vals = [plsc.load_gather(xs_v, [idxs[g]]) for g in range(ng)]
        return [
            jnp.where(vals[g] < qs[g], cnts[g] + s, cnts[g])
            for g in range(ng)
        ]

    def _bs_step(i, cnts):
        cnts = list(cnts)
        cnts = _bs_round(cnts, M >> (2 * i + 1))
        cnts = _bs_round(cnts, M >> (2 * i + 2))
        return tuple(cnts)

    los = list(lax.fori_loop(
        0, 7, _bs_step,
        tuple(jnp.zeros((L,), jnp.int32) for _ in range(ng))))
    lefts = [los[g] - 1 for g in range(ng)]
    rights = los
    dls, drs = [], []
    for g in range(ng):
        lval = plsc.load_gather(xs_v, [jnp.maximum(lefts[g], 0)])
        rval = plsc.load_gather(xs_v, [jnp.minimum(rights[g], M - 1)])
        dls.append(jnp.where(lefts[g] >= 0, jnp.abs(qs[g] - lval), BIG))
        drs.append(jnp.where(rights[g] < M, jnp.abs(qs[g] - rval), BIG))
        plsc.store_scatter(
            out_v, [lanes + g * L, jnp.zeros((L,), jnp.int32)], qs[g])
    def _merge_step(t, carry):
        lefts, dls, drs = (list(c) for c in carry)
        col = jnp.zeros((L,), jnp.int32) + t
        takes = [dls[g] <= drs[g] for g in range(ng)]
        for g in range(ng):
            plsc.store_scatter(
                out_v, [lanes + g * L, col],
                jnp.where(takes[g], dls[g], drs[g]))
        for g in range(ng):
            lefts[g] = jnp.where(takes[g], lefts[g] - 1, lefts[g])
        rights = [lefts[g] + (t + 1) for g in range(ng)]
        moved = [jnp.where(takes[g], lefts[g], rights[g]) for g in range(ng)]
        vals = [
            plsc.load_gather(xs_v, [jnp.clip(moved[g], 0, M - 1)])
            for g in range(ng)
        ]
        for g in range(ng):
            valid = jnp.where(takes[g], lefts[g] >= 0, rights[g] < M)
            nd = jnp.where(valid, jnp.abs(qs[g] - vals[g]), BIG)
            dls[g] = jnp.where(takes[g], nd, dls[g])
            drs[g] = jnp.where(takes[g], drs[g], nd)
        return tuple(lefts), tuple(dls), tuple(drs)

    _, dls, drs = lax.fori_loop(
        1, K, _merge_step,
        (tuple(lefts), tuple(dls), tuple(drs)))
    colk = jnp.zeros((L,), jnp.int32) + K
    for g in range(ng):
        plsc.store_scatter(
            out_v, [lanes + g * L, colk],
            jnp.where(dls[g] <= drs[g], dls[g], drs[g]))
    pltpu.sync_copy(out_v, out_hbm.at[pl.ds(base, QPW)])


@functools.lru_cache(maxsize=1)
def _make_sc_query():
    return pl.kernel(
        _sc_query_body,
        out_type=jax.ShapeDtypeStruct((N, K + 1), jnp.float32),
        mesh=plsc.VectorSubcoreMesh(
            core_axis_name="c", subcore_axis_name="s",
            num_cores=NC, num_subcores=NS),
        scratch_types=[
            pltpu.VMEM((M,), jnp.float32),
            pltpu.VMEM((QPW,), jnp.float32),
            pltpu.VMEM((QPW, K + 1), jnp.float32),
            pltpu.VMEM_SHARED((M,), jnp.float32),
            pltpu.SemaphoreType.DMA,
        ],
        compiler_params=pltpu.CompilerParams(needs_layout_passes=False),
    )


@jax.jit
def kernel(x, x_measured):
    xs = _tc_sort(x_measured.reshape(128, 128)).reshape(M)
    return _make_sc_query()(xs, x)
